# baseline (speedup 1.0000x reference)
import functools

import jax
import jax.numpy as jnp
from jax import lax
from jax.experimental import pallas as pl
from jax.experimental.pallas import tpu as pltpu
from jax.experimental.pallas import tpu_sc as plsc

VOCAB = 1000000
EMB = 64
B = 16384
CTX = 20

NC, NS = 2, 16
NW = NC * NS
EPW = B // NW
CB = 32
NCHUNK = EPW // CB
ROWS = CB * CTX
IDXW = 128
IDXROWS = ROWS // IDXW
NV = EMB // 16


def _log_sigmoid(x):
    e = jnp.exp(-jnp.abs(x))
    z = e / (e + 2.0)
    z2 = z * z
    p = 1.0 + z2 * ((1.0 / 3.0) + z2 * ((1.0 / 5.0) + z2 * ((1.0 / 7.0) + z2 * (1.0 / 9.0))))
    return jnp.minimum(x, 0.0) - 2.0 * z * p


def _cbow_body(pos_ctx, pos_tgt, neg_ctx, neg_tgt, u_emb, v_emb, out,
               idx_v, rows_v, tgt_v, vrows_v, usum_v, dots_v, out_v, gsem):
    wid = lax.axis_index("s") * NC + lax.axis_index("c")

    def chunk_body(c, _):
        base = wid * EPW + c * CB

        for side, (ctx_hbm, tgt_hbm) in enumerate(((pos_ctx, pos_tgt), (neg_ctx, neg_tgt))):
            pltpu.sync_copy(ctx_hbm.at[pl.ds(base * CTX, ROWS)], idx_v)
            copies = [
                pltpu.async_copy(u_emb.at[idx_v.at[pl.ds(j * IDXW, IDXW)]],
                                 rows_v.at[pl.ds(j * IDXW, IDXW)], gsem)
                for j in range(IDXROWS)
            ]
            pltpu.sync_copy(tgt_hbm.at[pl.ds(base, CB)], tgt_v)
            vcp = pltpu.async_copy(v_emb.at[tgt_v], vrows_v, gsem)
            for cp in copies:
                cp.wait()
            vcp.wait()

            def elem_body(i, _):
                rbase = i * CTX
                for d in range(NV):
                    acc = rows_v[rbase, pl.ds(d * 16, 16)]
                    for j in range(1, CTX):
                        acc = acc + rows_v[rbase + j, pl.ds(d * 16, 16)]
                    usum_v[i, pl.ds(d * 16, 16)] = acc
                return 0

            lax.fori_loop(0, CB, elem_body, 0)

            lanes = lax.iota(jnp.int32, 16)
            for g in range(CB // 16):
                idx_i = lanes + g * 16

                def dot_body(d, acc):
                    dcol = jnp.full((16,), d, jnp.int32)
                    u_d = plsc.load_gather(usum_v, [idx_i, dcol])
                    v_d = plsc.load_gather(vrows_v, [idx_i, dcol])
                    return acc + u_d * v_d

                dv = lax.fori_loop(0, EMB, dot_body, jnp.zeros((16,), jnp.float32))
                dots_v[side, pl.ds(g * 16, 16)] = dv

        for g in range(CB // 16):
            dp = dots_v[0, pl.ds(g * 16, 16)]
            dn = dots_v[1, pl.ds(g * 16, 16)]
            out_v[pl.ds(g * 16, 16)] = -(_log_sigmoid(dp) + _log_sigmoid(-dn))
        pltpu.sync_copy(out_v, out.at[pl.ds(base, CB)])
        return 0

    lax.fori_loop(0, NCHUNK, chunk_body, 0)


def kernel(pos_context_word_ids, pos_target_word_id,
           neg_context_word_ids, neg_target_word_id, u_emb, v_emb):
    pos_ctx = pos_context_word_ids.reshape(B * CTX)
    neg_ctx = neg_context_word_ids.reshape(B * CTX)
    pos_tgt = pos_target_word_id.reshape(B)
    neg_tgt = neg_target_word_id.reshape(B)

    mesh = plsc.VectorSubcoreMesh(core_axis_name="c", subcore_axis_name="s")
    run = functools.partial(
        pl.kernel,
        mesh=mesh,
        compiler_params=pltpu.CompilerParams(
            needs_layout_passes=False, use_tc_tiling_on_sc=False),
        out_type=jax.ShapeDtypeStruct((B,), jnp.float32),
        scratch_types=[
            pltpu.VMEM((ROWS,), jnp.int32),
            pltpu.VMEM((ROWS, EMB), jnp.float32),
            pltpu.VMEM((CB,), jnp.int32),
            pltpu.VMEM((CB, EMB), jnp.float32),
            pltpu.VMEM((CB, EMB), jnp.float32),
            pltpu.VMEM((2, CB), jnp.float32),
            pltpu.VMEM((CB,), jnp.float32),
            pltpu.SemaphoreType.DMA,
        ],
    )(_cbow_body)
    return run(pos_ctx, pos_tgt, neg_ctx, neg_tgt, u_emb, v_emb)
